# baseline (speedup 1.0000x reference)
import functools

import jax
import jax.numpy as jnp
from jax import lax
from jax.experimental import pallas as pl
from jax.experimental.pallas import tpu as pltpu
from jax.experimental.pallas import tpu_sc as plsc

_CHUNK = 8
_NBUF = 4
_NSP = 6


@functools.lru_cache(maxsize=None)
def _build(B: int, dim: int):
    info = plsc.get_sparse_core_info()
    nc, ns = info.num_cores, info.num_subcores
    nw = nc * ns
    b_per_w = B // nw
    g_total = b_per_w // _CHUNK
    assert (g_total - 8) % 12 == 0

    mesh = plsc.VectorSubcoreMesh(core_axis_name="c", subcore_axis_name="s")

    @functools.partial(
        pl.kernel,
        mesh=mesh,
        out_type=jax.ShapeDtypeStruct((B, dim), jnp.float32),
        scratch_types=[
            pltpu.VMEM((b_per_w,), jnp.int32),
            *[pltpu.VMEM((_CHUNK, dim), jnp.float32) for _ in range(_NBUF)],
            pltpu.VMEM_SHARED((ns, _NSP, _CHUNK, dim), jnp.float32),
            *[pltpu.SemaphoreType.DMA for _ in range(2 * _NBUF + _NSP)],
        ],
    )
    def k(pos_hbm, table_hbm, out_hbm, idx_v, *rest):
        bufs = rest[:_NBUF]
        sp = rest[_NBUF]
        asem = rest[_NBUF + 1 : _NBUF + 1 + _NBUF]
        bsem = rest[_NBUF + 1 + _NBUF : _NBUF + 1 + 2 * _NBUF]
        csem = rest[_NBUF + 1 + 2 * _NBUF :]

        cid = lax.axis_index("c")
        sid = lax.axis_index("s")
        wid = sid * nc + cid
        base = wid * b_per_w
        pltpu.sync_copy(pos_hbm.at[pl.ds(base, b_per_w)], idx_v)

        def A(g, b):
            return pltpu.make_async_copy(
                table_hbm.at[idx_v.at[pl.ds(g * _CHUNK, _CHUNK)]],
                bufs[b],
                asem[b],
            )

        def Bc(g, b, sl):
            return pltpu.make_async_copy(bufs[b], sp.at[sid, sl], bsem[b])

        def Cc(g, sl):
            return pltpu.make_async_copy(
                sp.at[sid, sl],
                out_hbm.at[pl.ds(base + g * _CHUNK, _CHUNK)],
                csem[sl],
            )

        def step(g):
            A(g, g % _NBUF).wait()
            if g >= 4:
                Cc(g - 4, (g - 4) % _NSP).wait()
            Bc(g, g % _NBUF, g % _NSP).start()
            if g >= 2:
                Bc(g - 2, (g - 2) % _NBUF, (g - 2) % _NSP).wait()
                Cc(g - 2, (g - 2) % _NSP).start()
            if g + 2 < g_total:
                A(g + 2, (g + 2) % _NBUF).start()

        A(0, 0).start()
        A(1, 1).start()
        for g in range(4):
            step(g)

        def body(o, carry):
            for j in range(12):
                g = o * 12 + j + 4
                A(g, (j + 4) % _NBUF).wait()
                Cc(g - 4, j % _NSP).wait()
                Bc(g, (j + 4) % _NBUF, (j + 4) % _NSP).start()
                Bc(g - 2, (j + 2) % _NBUF, (j + 2) % _NSP).wait()
                Cc(g - 2, (j + 2) % _NSP).start()
                A(g + 2, (j + 6) % _NBUF).start()
            return carry

        lax.fori_loop(0, (g_total - 8) // 12, body, 0)

        for g in range(g_total - 4, g_total):
            step(g)

        Bc(g_total - 2, (g_total - 2) % _NBUF, (g_total - 2) % _NSP).wait()
        Cc(g_total - 2, (g_total - 2) % _NSP).start()
        Bc(g_total - 1, (g_total - 1) % _NBUF, (g_total - 1) % _NSP).wait()
        Cc(g_total - 1, (g_total - 1) % _NSP).start()
        for g in range(g_total - 4, g_total):
            Cc(g, g % _NSP).wait()

    return k


def kernel(pos, pe_weight):
    b, s = pos.shape
    idx = pos.reshape(-1).astype(jnp.int32)
    out = _build(b * s, pe_weight.shape[1])(idx, pe_weight)
    return out.reshape(b, s, pe_weight.shape[1])


# speedup vs baseline: 1.0481x; 1.0481x over previous
import functools

import jax
import jax.numpy as jnp
from jax import lax
from jax.experimental import pallas as pl
from jax.experimental.pallas import tpu as pltpu
from jax.experimental.pallas import tpu_sc as plsc

_CHUNK = 8
_NBUF = 8


@functools.lru_cache(maxsize=None)
def _build(B: int, dim: int):
    info = plsc.get_sparse_core_info()
    nc, ns = info.num_cores, info.num_subcores
    nw = nc * ns
    assert B % (nw * _CHUNK * _NBUF) == 0
    b_per_w = B // nw
    g_total = b_per_w // _CHUNK

    mesh = plsc.VectorSubcoreMesh(core_axis_name="c", subcore_axis_name="s")

    @functools.partial(
        pl.kernel,
        mesh=mesh,
        out_type=jax.ShapeDtypeStruct((B, dim), jnp.float32),
        scratch_types=[
            pltpu.VMEM((b_per_w,), jnp.int32),
            *[pltpu.VMEM((_CHUNK, dim), jnp.float32) for _ in range(_NBUF)],
            *[pltpu.SemaphoreType.DMA for _ in range(2 * _NBUF)],
        ],
    )
    def k(pos_hbm, table_hbm, out_hbm, idx_v, *rest):
        bufs = rest[:_NBUF]
        gsem = rest[_NBUF : 2 * _NBUF]
        ssem = rest[2 * _NBUF :]

        wid = lax.axis_index("s") * nc + lax.axis_index("c")
        base = wid * b_per_w
        pltpu.sync_copy(pos_hbm.at[pl.ds(base, b_per_w)], idx_v)

        def gather(g, b):
            return pltpu.make_async_copy(
                table_hbm.at[idx_v.at[pl.ds(g * _CHUNK, _CHUNK)]],
                bufs[b],
                gsem[b],
            )

        def scatter(g, b):
            return pltpu.make_async_copy(
                bufs[b],
                out_hbm.at[pl.ds(base + g * _CHUNK, _CHUNK)],
                ssem[b],
            )

        for b in range(_NBUF):
            gather(b, b).start()

        def body(o, carry):
            for b in range(_NBUF):
                g = o * _NBUF + b
                gather(g, b).wait()
                scatter(g, b).start()
                scatter(g, b).wait()
                gather(g + _NBUF, b).start()
            return carry

        lax.fori_loop(0, g_total // _NBUF - 1, body, 0)

        for b in range(_NBUF):
            g = g_total - _NBUF + b
            gather(g, b).wait()
            scatter(g, b).start()
        for b in range(_NBUF):
            scatter(g_total - _NBUF + b, b).wait()

    return k


def kernel(pos, pe_weight):
    b, s = pos.shape
    idx = pos.reshape(-1).astype(jnp.int32)
    out = _build(b * s, pe_weight.shape[1])(idx, pe_weight)
    return out.reshape(b, s, pe_weight.shape[1])
